# baseline (speedup 1.0000x reference)
import jax
import jax.numpy as jnp
from jax import lax
from jax.experimental import pallas as pl
from jax.experimental.pallas import tpu as pltpu
from jax.experimental.pallas import tpu_sc as plsc

VOCAB = 1000000
EMB = 64
B = 1024
S = 200
EPS = 1e-6

N = B * S
NC, NS, L = 2, 16, 16
NW = NC * NS
ROWS_W = B // NW
PER_W = N // NW
CHUNK = 128
NCHUNK = PER_W // CHUNK
NBUF = 4
RUNROLL = 2


def _rsqrt_f32(x):
    i = lax.bitcast_convert_type(x, jnp.int32)
    i = jnp.int32(0x5F3759DF) - lax.shift_right_logical(i, 1)
    y = lax.bitcast_convert_type(i, jnp.float32)
    for _ in range(3):
        y = y * (1.5 - 0.5 * x * y * y)
    return y


def _sc_body(table_hbm, src_hbm, gam_hbm, bet_hbm, out_hbm,
             idx_v, buf_v, gam_v, bet_v, gsem, osem):
    wid = lax.axis_index("s") * NC + lax.axis_index("c")
    for i in range(ROWS_W):
        pltpu.async_copy(src_hbm.at[wid * ROWS_W + i],
                         idx_v.at[pl.ds(i * S, S)], gsem.at[0])
    for i in range(ROWS_W):
        pltpu.make_async_copy(src_hbm.at[0], idx_v.at[pl.ds(0, S)],
                              gsem.at[0]).wait()
    pltpu.sync_copy(gam_hbm, gam_v)
    pltpu.sync_copy(bet_hbm, bet_v)

    gam = [gam_v[pl.ds(k * L, L)] for k in range(EMB // L)]
    bet = [bet_v[pl.ds(k * L, L)] for k in range(EMB // L)]
    inv = jnp.full((L,), 1.0 / EMB, jnp.float32)

    def issue_gather(c):
        sl = c & (NBUF - 1)
        idx_ref = idx_v.at[pl.ds(c * CHUNK, CHUNK)]
        pltpu.async_copy(table_hbm.at[idx_ref], buf_v.at[sl], gsem.at[sl])

    def prologue(c, carry):
        issue_gather(c)
        return carry

    lax.fori_loop(0, NBUF - 1, prologue, 0)

    def chunk_body(g, carry):
        slot = g & (NBUF - 1)
        nxt = g + NBUF - 1

        @pl.when(nxt < NCHUNK)
        def _():
            @pl.when(g >= 1)
            def _():
                pltpu.make_async_copy(
                    buf_v.at[nxt & (NBUF - 1)],
                    out_hbm.at[pl.ds(0, CHUNK)],
                    osem.at[nxt & (NBUF - 1)]).wait()
            issue_gather(nxt)

        pltpu.make_async_copy(table_hbm.at[idx_v.at[pl.ds(0, CHUNK)]],
                              buf_v.at[slot], gsem.at[slot]).wait()

        def row_body(ri, c2):
            for u in range(RUNROLL):
                r = ri * RUNROLL + u
                v = [buf_v[slot, r, pl.ds(k * L, L)] for k in range(EMB // L)]
                s = (v[0] + v[1]) + (v[2] + v[3])
                q = (v[0] * v[0] + v[1] * v[1]) + (v[2] * v[2] + v[3] * v[3])
                ssum = jnp.broadcast_to(jnp.sum(s), (L,))
                qsum = jnp.broadcast_to(jnp.sum(q), (L,))
                mu = ssum * inv
                var = qsum * inv - mu * mu
                rstd = _rsqrt_f32(var + EPS)
                for k in range(EMB // L):
                    o = (v[k] - mu) * rstd * gam[k] + bet[k]
                    buf_v[slot, r, pl.ds(k * L, L)] = o
            return c2

        lax.fori_loop(0, CHUNK // RUNROLL, row_body, 0)
        pltpu.async_copy(buf_v.at[slot],
                         out_hbm.at[pl.ds(wid * PER_W + g * CHUNK, CHUNK)],
                         osem.at[slot])
        return carry

    lax.fori_loop(0, NCHUNK, chunk_body, 0)

    for last in range(NCHUNK - NBUF + 1, NCHUNK):
        pltpu.make_async_copy(
            buf_v.at[last & (NBUF - 1)],
            out_hbm.at[pl.ds(wid * PER_W + last * CHUNK, CHUNK)],
            osem.at[last & (NBUF - 1)]).wait()


@jax.jit
def _embed_ln(table, src_i32, gamma, beta):
    mesh = plsc.VectorSubcoreMesh(core_axis_name="c", subcore_axis_name="s")
    return pl.kernel(
        _sc_body,
        out_type=jax.ShapeDtypeStruct((N, EMB), jnp.float32),
        mesh=mesh,
        compiler_params=pltpu.CompilerParams(
            needs_layout_passes=False, use_tc_tiling_on_sc=False),
        scratch_types=[
            pltpu.VMEM((PER_W,), jnp.int32),
            pltpu.VMEM((NBUF, CHUNK, EMB), jnp.float32),
            pltpu.VMEM((EMB,), jnp.float32),
            pltpu.VMEM((EMB,), jnp.float32),
            pltpu.SemaphoreType.DMA((NBUF,)),
            pltpu.SemaphoreType.DMA((NBUF,)),
        ],
    )(table, src_i32, gamma, beta)


def kernel(src, seg, table, gamma, beta):
    del seg
    out = _embed_ln(table, src.astype(jnp.int32), gamma, beta)
    return out.reshape(B, S, EMB)


# speedup vs baseline: 1.1657x; 1.1657x over previous
import jax
import jax.numpy as jnp
from jax import lax
from jax.experimental import pallas as pl
from jax.experimental.pallas import tpu as pltpu
from jax.experimental.pallas import tpu_sc as plsc

VOCAB = 1000000
EMB = 64
B = 1024
S = 200
EPS = 1e-6

N = B * S
NC, NS, L = 2, 16, 16
NW = NC * NS
PER_W = N // NW
CHUNK = 128
NCHUNK = PER_W // CHUNK
NBUF = 4

BLKP = 2048


def _sc_gather_body(table_hbm, idx_hbm, out_hbm, idx_v, buf_v, gsem, osem):
    wid = lax.axis_index("s") * NC + lax.axis_index("c")
    pltpu.sync_copy(idx_hbm.at[pl.ds(wid * NCHUNK, NCHUNK)], idx_v)

    def issue_gather(c):
        sl = c & (NBUF - 1)
        pltpu.async_copy(table_hbm.at[idx_v.at[c]], buf_v.at[sl], gsem.at[sl])

    def prologue(c, carry):
        issue_gather(c)
        return carry

    lax.fori_loop(0, NBUF - 1, prologue, 0)

    def chunk_body(g, carry):
        slot = g & (NBUF - 1)
        nxt = g + NBUF - 1

        @pl.when(nxt < NCHUNK)
        def _():
            @pl.when(g >= 1)
            def _():
                pltpu.make_async_copy(
                    buf_v.at[nxt & (NBUF - 1)],
                    out_hbm.at[pl.ds(0, CHUNK)],
                    osem.at[nxt & (NBUF - 1)]).wait()
            issue_gather(nxt)

        pltpu.make_async_copy(table_hbm.at[idx_v.at[g]], buf_v.at[slot],
                              gsem.at[slot]).wait()
        pltpu.async_copy(buf_v.at[slot],
                         out_hbm.at[pl.ds(wid * PER_W + g * CHUNK, CHUNK)],
                         osem.at[slot])
        return carry

    lax.fori_loop(0, NCHUNK, chunk_body, 0)

    for last in range(NCHUNK - NBUF + 1, NCHUNK):
        pltpu.make_async_copy(
            buf_v.at[last & (NBUF - 1)],
            out_hbm.at[pl.ds(wid * PER_W + last * CHUNK, CHUNK)],
            osem.at[last & (NBUF - 1)]).wait()


def _tc_ln_body(x_ref, gam_ref, bet_ref, o_ref):
    for h in (0, 1):
        y = x_ref[:, pl.ds(h * EMB, EMB)]
        mu = jnp.mean(y, axis=-1, keepdims=True)
        d = y - mu
        var = jnp.mean(d * d, axis=-1, keepdims=True)
        o_ref[:, pl.ds(h * EMB, EMB)] = (
            gam_ref[:, pl.ds(h * EMB, EMB)] * d * lax.rsqrt(var + EPS)
            + bet_ref[:, pl.ds(h * EMB, EMB)])


@jax.jit
def _embed_ln(table, idx2d, gamma, beta):
    mesh = plsc.VectorSubcoreMesh(core_axis_name="c", subcore_axis_name="s")
    gathered = pl.kernel(
        _sc_gather_body,
        out_type=jax.ShapeDtypeStruct((N, EMB), jnp.float32),
        mesh=mesh,
        compiler_params=pltpu.CompilerParams(
            needs_layout_passes=False, use_tc_tiling_on_sc=False),
        scratch_types=[
            pltpu.VMEM((NCHUNK, CHUNK), jnp.int32),
            pltpu.VMEM((NBUF, CHUNK, EMB), jnp.float32),
            pltpu.SemaphoreType.DMA((NBUF,)),
            pltpu.SemaphoreType.DMA((NBUF,)),
        ],
    )(table, idx2d)

    paired = gathered.reshape(N // 2, 2 * EMB)
    gam2 = jnp.tile(gamma, 2).reshape(1, 2 * EMB)
    bet2 = jnp.tile(beta, 2).reshape(1, 2 * EMB)
    out = pl.pallas_call(
        _tc_ln_body,
        out_shape=jax.ShapeDtypeStruct((N // 2, 2 * EMB), jnp.float32),
        grid=(N // 2 // BLKP,),
        in_specs=[
            pl.BlockSpec((BLKP, 2 * EMB), lambda i: (i, 0)),
            pl.BlockSpec((1, 2 * EMB), lambda i: (0, 0)),
            pl.BlockSpec((1, 2 * EMB), lambda i: (0, 0)),
        ],
        out_specs=pl.BlockSpec((BLKP, 2 * EMB), lambda i: (i, 0)),
    )(paired, gam2, bet2)
    return out


def kernel(src, seg, table, gamma, beta):
    del seg
    idx2d = src.astype(jnp.int32).reshape(NW * NCHUNK, CHUNK)
    out = _embed_ln(table, idx2d, gamma, beta)
    return out.reshape(B, S, EMB)


# speedup vs baseline: 1.2814x; 1.0992x over previous
import jax
import jax.numpy as jnp
from jax import lax
from jax.experimental import pallas as pl
from jax.experimental.pallas import tpu as pltpu
from jax.experimental.pallas import tpu_sc as plsc

VOCAB = 1000000
EMB = 64
PAIR = 128
B = 1024
S = 200
EPS = 1e-6

N = B * S
NC, NS, L = 2, 16, 16
NW = NC * NS
PER_W = N // NW
CHUNK = 128
NCHUNK = PER_W // CHUNK
NCHUNK_PAD = 56
NBUF = 4

BLK = 2048


def _sc_gather_body(table_hbm, idx_hbm, out_hbm, idx_v, buf_v, gsem, osem):
    wid = lax.axis_index("s") * NC + lax.axis_index("c")
    pltpu.sync_copy(idx_hbm.at[wid], idx_v)

    def issue_gather(c):
        sl = c & (NBUF - 1)
        pltpu.async_copy(table_hbm.at[idx_v.at[c]], buf_v.at[sl], gsem.at[sl])

    def prologue(c, carry):
        issue_gather(c)
        return carry

    lax.fori_loop(0, NBUF - 1, prologue, 0)

    def chunk_body(g, carry):
        slot = g & (NBUF - 1)
        nxt = g + NBUF - 1

        @pl.when(nxt < NCHUNK)
        def _():
            @pl.when(g >= 1)
            def _():
                pltpu.make_async_copy(
                    buf_v.at[nxt & (NBUF - 1)],
                    out_hbm.at[pl.ds(0, CHUNK)],
                    osem.at[nxt & (NBUF - 1)]).wait()
            issue_gather(nxt)

        pltpu.make_async_copy(table_hbm.at[idx_v.at[g]], buf_v.at[slot],
                              gsem.at[slot]).wait()
        pltpu.async_copy(buf_v.at[slot],
                         out_hbm.at[pl.ds(wid * PER_W + g * CHUNK, CHUNK)],
                         osem.at[slot])
        return carry

    lax.fori_loop(0, NCHUNK, chunk_body, 0)

    for last in range(NCHUNK - NBUF + 1, NCHUNK):
        pltpu.make_async_copy(
            buf_v.at[last & (NBUF - 1)],
            out_hbm.at[pl.ds(wid * PER_W + last * CHUNK, CHUNK)],
            osem.at[last & (NBUF - 1)]).wait()


def _tc_ln_body(x_ref, gam_ref, bet_ref, o_ref):
    y = x_ref[:, pl.ds(0, EMB)]
    mu = jnp.mean(y, axis=-1, keepdims=True)
    d = y - mu
    var = jnp.mean(d * d, axis=-1, keepdims=True)
    o_ref[...] = gam_ref[...] * d * lax.rsqrt(var + EPS) + bet_ref[...]


@jax.jit
def _embed_ln(table_pad, idx3d, gamma, beta):
    mesh = plsc.VectorSubcoreMesh(core_axis_name="c", subcore_axis_name="s")
    gathered = pl.kernel(
        _sc_gather_body,
        out_type=jax.ShapeDtypeStruct((N, PAIR), jnp.float32),
        mesh=mesh,
        compiler_params=pltpu.CompilerParams(
            needs_layout_passes=False, use_tc_tiling_on_sc=True),
        scratch_types=[
            pltpu.VMEM((NCHUNK_PAD, CHUNK), jnp.int32),
            pltpu.VMEM((NBUF, CHUNK, PAIR), jnp.float32),
            pltpu.SemaphoreType.DMA((NBUF,)),
            pltpu.SemaphoreType.DMA((NBUF,)),
        ],
    )(table_pad, idx3d)

    out = pl.pallas_call(
        _tc_ln_body,
        out_shape=jax.ShapeDtypeStruct((N, EMB), jnp.float32),
        grid=(N // BLK,),
        in_specs=[
            pl.BlockSpec((BLK, PAIR), lambda i: (i, 0)),
            pl.BlockSpec((1, EMB), lambda i: (0, 0)),
            pl.BlockSpec((1, EMB), lambda i: (0, 0)),
        ],
        out_specs=pl.BlockSpec((BLK, EMB), lambda i: (i, 0)),
    )(gathered, gamma.reshape(1, EMB), beta.reshape(1, EMB))
    return out


def kernel(src, seg, table, gamma, beta):
    del seg
    table_pad = jnp.pad(table, ((0, 0), (0, PAIR - EMB)))
    idx3d = src.astype(jnp.int32).reshape(NW, NCHUNK, CHUNK)
    idx3d = jnp.pad(idx3d, ((0, 0), (0, NCHUNK_PAD - NCHUNK), (0, 0)))
    out = _embed_ln(table_pad, idx3d, gamma, beta)
    return out.reshape(B, S, EMB)
